# baseline (speedup 1.0000x reference)
import functools

import jax
import jax.numpy as jnp
from jax import lax
from jax.experimental import pallas as pl
from jax.experimental.pallas import tpu as pltpu
from jax.experimental.pallas import tpu_sc as plsc

N_USERS = 100000
N_ENTITIES = 100000
N_RELATIONS = 32
EMB_DIM = 64
KGE_DIM = 64
B = 16384

_NC = 2
_NS = 16
_NW = _NC * _NS
_TOT = 3 * B
_PER_W = _TOT // _NW
_CH = 128
_NCHUNK = _PER_W // _CH


def _sc_gather_body(user_hbm, ent_hbm, idx_hbm, out_u, out_e,
                    idxv, uidxv, eidxv, rows_u, rows_e, sem_u, sem_e):
    wid = lax.axis_index("s") * _NC + lax.axis_index("c")
    base_w = wid * _PER_W

    @pl.loop(0, _NCHUNK)
    def _chunk(ci):
        base = base_w + ci * _CH
        pltpu.sync_copy(idx_hbm.at[pl.ds(base, _CH)], idxv)

        @pl.loop(0, _CH, step=16)
        def _mask(j):
            v = idxv[pl.ds(j, 16)]
            m = v < N_USERS
            uidxv[pl.ds(j, 16)] = jnp.where(m, v, 0)
            eidxv[pl.ds(j, 16)] = jnp.where(m, 0, v - N_USERS)

        cu = pltpu.async_copy(user_hbm.at[uidxv], rows_u, sem_u)
        ce = pltpu.async_copy(ent_hbm.at[eidxv], rows_e, sem_e)
        cu.wait()
        ce.wait()
        pltpu.sync_copy(rows_u, out_u.at[pl.ds(base, _CH)])
        pltpu.sync_copy(rows_e, out_e.at[pl.ds(base, _CH)])


def _sc_gather(user_embed, entity_embed, idx_all):
    mesh = plsc.VectorSubcoreMesh(core_axis_name="c", subcore_axis_name="s")
    f32 = jnp.float32
    kern = pl.kernel(
        _sc_gather_body,
        mesh=mesh,
        compiler_params=pltpu.CompilerParams(use_tc_tiling_on_sc=False),
        out_type=[jax.ShapeDtypeStruct((_TOT, EMB_DIM), f32),
                  jax.ShapeDtypeStruct((_TOT, EMB_DIM), f32)],
        scratch_types=[
            pltpu.VMEM((_CH,), jnp.int32),
            pltpu.VMEM((_CH,), jnp.int32),
            pltpu.VMEM((_CH,), jnp.int32),
            pltpu.VMEM((_CH, EMB_DIM), f32),
            pltpu.VMEM((_CH, EMB_DIM), f32),
            pltpu.SemaphoreType.DMA,
            pltpu.SemaphoreType.DMA,
        ],
    )
    return kern(user_embed, entity_embed, idx_all)


_T = 512
_GRID = B // _T


def _tc_body(hu, he, pu, pe, nu, ne, hidx, pidx, nidx, rel,
             rel_emb, w2, out_ref):
    f32 = jnp.float32
    h = jnp.where(hidx[...] < N_USERS, hu[...], he[...])
    p = jnp.where(pidx[...] < N_USERS, pu[...], pe[...])
    n = jnp.where(nidx[...] < N_USERS, nu[...], ne[...])
    dp = h - p
    dn = h - n
    x = jnp.concatenate([dp, dn], axis=0)
    y = lax.dot_general(x, w2[...], (((1,), (0,)), ((), ())),
                        preferred_element_type=f32,
                        precision=lax.Precision.HIGHEST)
    oh = (rel[...] == lax.broadcasted_iota(jnp.int32, (_T, N_RELATIONS), 1)
          ).astype(f32)
    r_e = lax.dot_general(oh, rel_emb[...], (((1,), (0,)), ((), ())),
                          preferred_element_type=f32,
                          precision=lax.Precision.HIGHEST)
    oh2 = jnp.concatenate([oh, oh], axis=0)
    acc = jnp.zeros((2 * _T, KGE_DIM), f32)
    for r in range(N_RELATIONS):
        acc = acc + y[:, r * KGE_DIM:(r + 1) * KGE_DIM] * oh2[:, r:r + 1]
    sp = acc[:_T] + r_e
    sn = acc[_T:] + r_e
    ps = jnp.sum(sp * sp, axis=1)
    ns = jnp.sum(sn * sn, axis=1)
    d = ps - ns
    part = jnp.sum(jnp.maximum(d, 0.0) + jnp.log1p(jnp.exp(-jnp.abs(d))))

    @pl.when(pl.program_id(0) == 0)
    def _init():
        out_ref[0, 0] = 0.0

    out_ref[0, 0] += part * (1.0 / B)


def _tc_loss(rows_u, rows_e, heads, pos_tails, neg_tails, relations,
             relation_embed, w2, interpret=False):
    f32 = jnp.float32
    row_spec = lambda seg: pl.BlockSpec((_T, EMB_DIM),
                                        lambda i, s=seg: (s * _GRID + i, 0))
    idx_spec = pl.BlockSpec((_T, 1), lambda i: (i, 0))
    full = lambda shape: pl.BlockSpec(shape, lambda i: (0, 0))
    out = pl.pallas_call(
        _tc_body,
        grid=(_GRID,),
        in_specs=[
            row_spec(0), row_spec(0),
            row_spec(1), row_spec(1),
            row_spec(2), row_spec(2),
            idx_spec, idx_spec, idx_spec, idx_spec,
            full((N_RELATIONS, KGE_DIM)),
            full((EMB_DIM, N_RELATIONS * KGE_DIM)),
        ],
        out_specs=pl.BlockSpec(memory_space=pltpu.SMEM),
        out_shape=jax.ShapeDtypeStruct((1, 1), f32),
        interpret=interpret,
    )(rows_u, rows_e, rows_u, rows_e, rows_u, rows_e,
      heads, pos_tails, neg_tails, relations,
      relation_embed, w2)
    return out[0, 0]


def kernel(user_embed, entity_embed, relation_embed, trans_W,
           heads, relations, pos_tails, neg_tails):
    i32 = jnp.int32
    heads = heads.astype(i32)
    pos_tails = pos_tails.astype(i32)
    neg_tails = neg_tails.astype(i32)
    relations = relations.astype(i32)

    idx_all = jnp.concatenate([heads, pos_tails, neg_tails])
    rows_u, rows_e = _sc_gather(user_embed, entity_embed, idx_all)

    w2 = jnp.transpose(trans_W, (1, 0, 2)).reshape(EMB_DIM,
                                                   N_RELATIONS * KGE_DIM)
    return _tc_loss(rows_u, rows_e,
                    heads.reshape(B, 1), pos_tails.reshape(B, 1),
                    neg_tails.reshape(B, 1), relations.reshape(B, 1),
                    relation_embed, w2)


# speedup vs baseline: 1.1132x; 1.1132x over previous
import functools

import jax
import jax.numpy as jnp
from jax import lax
from jax.experimental import pallas as pl
from jax.experimental.pallas import tpu as pltpu
from jax.experimental.pallas import tpu_sc as plsc

N_USERS = 100000
N_ENTITIES = 100000
N_RELATIONS = 32
EMB_DIM = 64
KGE_DIM = 64
B = 16384

_NC = 2
_NS = 16
_NW = _NC * _NS
_TOT = 3 * B
_PER_W = _TOT // _NW
_CH = 128
_NCHUNK = _PER_W // _CH


def _sc_gather_body(user_hbm, ent_hbm, idx_hbm, out_u, out_e, *sc):
    idxv = sc[0:2]
    uidx = sc[2:4]
    eidx = sc[4:6]
    ru = sc[6:8]
    re = sc[8:10]
    si = sc[10:12]
    sgu = sc[12:14]
    sge = sc[14:16]
    swu = sc[16:18]
    swe = sc[18:20]
    wid = lax.axis_index("s") * _NC + lax.axis_index("c")
    base_w = wid * _PER_W

    def idx_slice(i):
        return idx_hbm.at[pl.ds(base_w + i * _CH, _CH)]

    def out_slice(ref, i):
        return ref.at[pl.ds(base_w + i * _CH, _CH)]

    idx_cp = [None] * (_NCHUNK + 2)
    g_u = [None] * _NCHUNK
    g_e = [None] * _NCHUNK
    wb_u = [None] * _NCHUNK
    wb_e = [None] * _NCHUNK
    idx_cp[0] = pltpu.async_copy(idx_slice(0), idxv[0], si[0])
    idx_cp[1] = pltpu.async_copy(idx_slice(1), idxv[1], si[1])
    for i in range(_NCHUNK):
        b = i % 2
        idx_cp[i].wait()

        @pl.loop(0, _CH, step=16)
        def _mask(j, b=b):
            v = idxv[b][pl.ds(j, 16)]
            m = v < N_USERS
            uidx[b][pl.ds(j, 16)] = jnp.where(m, v, 0)
            eidx[b][pl.ds(j, 16)] = jnp.where(m, 0, v - N_USERS)

        if i + 2 < _NCHUNK:
            idx_cp[i + 2] = pltpu.async_copy(idx_slice(i + 2), idxv[b], si[b])
        if i >= 2:
            wb_u[i - 2].wait()
            wb_e[i - 2].wait()
        g_u[i] = pltpu.async_copy(user_hbm.at[uidx[b]], ru[b], sgu[b])
        g_e[i] = pltpu.async_copy(ent_hbm.at[eidx[b]], re[b], sge[b])
        if i >= 1:
            pb = (i - 1) % 2
            g_u[i - 1].wait()
            g_e[i - 1].wait()
            wb_u[i - 1] = pltpu.async_copy(ru[pb], out_slice(out_u, i - 1),
                                           swu[pb])
            wb_e[i - 1] = pltpu.async_copy(re[pb], out_slice(out_e, i - 1),
                                           swe[pb])
    last = _NCHUNK - 1
    lb = last % 2
    g_u[last].wait()
    g_e[last].wait()
    wb_u[last] = pltpu.async_copy(ru[lb], out_slice(out_u, last), swu[lb])
    wb_e[last] = pltpu.async_copy(re[lb], out_slice(out_e, last), swe[lb])
    wb_u[last - 1].wait()
    wb_e[last - 1].wait()
    wb_u[last].wait()
    wb_e[last].wait()


def _sc_gather(user_embed, entity_embed, idx_all):
    mesh = plsc.VectorSubcoreMesh(core_axis_name="c", subcore_axis_name="s")
    f32 = jnp.float32
    i32 = jnp.int32
    kern = pl.kernel(
        _sc_gather_body,
        mesh=mesh,
        compiler_params=pltpu.CompilerParams(use_tc_tiling_on_sc=False),
        out_type=[jax.ShapeDtypeStruct((_TOT, EMB_DIM), f32),
                  jax.ShapeDtypeStruct((_TOT, EMB_DIM), f32)],
        scratch_types=(
            [pltpu.VMEM((_CH,), i32) for _ in range(6)]
            + [pltpu.VMEM((_CH, EMB_DIM), f32) for _ in range(4)]
            + [pltpu.SemaphoreType.DMA for _ in range(10)]
        ),
    )
    return kern(user_embed, entity_embed, idx_all)


_T = 512
_GRID = B // _T


def _tc_body(hu, he, pu, pe, nu, ne, hidx, pidx, nidx, rel,
             rel_emb, w2, out_ref):
    f32 = jnp.float32
    h = jnp.where(hidx[...] < N_USERS, hu[...], he[...])
    p = jnp.where(pidx[...] < N_USERS, pu[...], pe[...])
    n = jnp.where(nidx[...] < N_USERS, nu[...], ne[...])
    dp = h - p
    dn = h - n
    x = jnp.concatenate([dp, dn], axis=0)
    y = lax.dot_general(x, w2[...], (((1,), (0,)), ((), ())),
                        preferred_element_type=f32,
                        precision=lax.Precision.DEFAULT)
    oh = (rel[...] == lax.broadcasted_iota(jnp.int32, (_T, N_RELATIONS), 1)
          ).astype(f32)
    r_e = lax.dot_general(oh, rel_emb[...], (((1,), (0,)), ((), ())),
                          preferred_element_type=f32,
                          precision=lax.Precision.HIGHEST)
    oh2 = jnp.concatenate([oh, oh], axis=0)
    acc = jnp.zeros((2 * _T, KGE_DIM), f32)
    for r in range(N_RELATIONS):
        acc = acc + y[:, r * KGE_DIM:(r + 1) * KGE_DIM] * oh2[:, r:r + 1]
    sp = acc[:_T] + r_e
    sn = acc[_T:] + r_e
    ps = jnp.sum(sp * sp, axis=1)
    ns = jnp.sum(sn * sn, axis=1)
    d = ps - ns
    part = jnp.sum(jnp.maximum(d, 0.0) + jnp.log1p(jnp.exp(-jnp.abs(d))))

    @pl.when(pl.program_id(0) == 0)
    def _init():
        out_ref[0, 0] = 0.0

    out_ref[0, 0] += part * (1.0 / B)


def _tc_loss(rows_u, rows_e, heads, pos_tails, neg_tails, relations,
             relation_embed, w2, interpret=False):
    f32 = jnp.float32
    row_spec = lambda seg: pl.BlockSpec((_T, EMB_DIM),
                                        lambda i, s=seg: (s * _GRID + i, 0))
    idx_spec = pl.BlockSpec((_T, 1), lambda i: (i, 0))
    full = lambda shape: pl.BlockSpec(shape, lambda i: (0, 0))
    out = pl.pallas_call(
        _tc_body,
        grid=(_GRID,),
        in_specs=[
            row_spec(0), row_spec(0),
            row_spec(1), row_spec(1),
            row_spec(2), row_spec(2),
            idx_spec, idx_spec, idx_spec, idx_spec,
            full((N_RELATIONS, KGE_DIM)),
            full((EMB_DIM, N_RELATIONS * KGE_DIM)),
        ],
        out_specs=pl.BlockSpec(memory_space=pltpu.SMEM),
        out_shape=jax.ShapeDtypeStruct((1, 1), f32),
        interpret=interpret,
    )(rows_u, rows_e, rows_u, rows_e, rows_u, rows_e,
      heads, pos_tails, neg_tails, relations,
      relation_embed, w2)
    return out[0, 0]


def kernel(user_embed, entity_embed, relation_embed, trans_W,
           heads, relations, pos_tails, neg_tails):
    i32 = jnp.int32
    heads = heads.astype(i32)
    pos_tails = pos_tails.astype(i32)
    neg_tails = neg_tails.astype(i32)
    relations = relations.astype(i32)

    idx_all = jnp.concatenate([heads, pos_tails, neg_tails])
    rows_u, rows_e = _sc_gather(user_embed, entity_embed, idx_all)

    w2 = jnp.transpose(trans_W, (1, 0, 2)).reshape(EMB_DIM,
                                                   N_RELATIONS * KGE_DIM)
    return _tc_loss(rows_u, rows_e,
                    heads.reshape(B, 1), pos_tails.reshape(B, 1),
                    neg_tails.reshape(B, 1), relations.reshape(B, 1),
                    relation_embed, w2)


# speedup vs baseline: 2.6322x; 2.3647x over previous
import functools

import jax
import jax.numpy as jnp
from jax import lax
from jax.experimental import pallas as pl
from jax.experimental.pallas import tpu as pltpu
from jax.experimental.pallas import tpu_sc as plsc

N_USERS = 100000
N_ENTITIES = 100000
N_RELATIONS = 32
EMB_DIM = 64
KGE_DIM = 64
B = 16384

_NC = 2
_NS = 16
_NW = _NC * _NS
_TOT = 3 * B
_PER_W = _TOT // _NW
_CH = 128
_NCHUNK = _PER_W // _CH


def _sc_gather_body(user_hbm, ent_hbm, idx_hbm, out_u, out_e, *sc):
    idxv = sc[0:2]
    fidx = sc[2:4]
    ru = sc[4:6]
    re = sc[6:8]
    si = sc[8:10]
    sgu = sc[10:12]
    sge = sc[12:14]
    swu = sc[14:16]
    swe = sc[16:18]
    wid = lax.axis_index("s") * _NC + lax.axis_index("c")
    base_w = wid * _PER_W

    def idx_slice(i):
        return idx_hbm.at[pl.ds(base_w + i * _CH, _CH)]

    def out_slice(ref, i):
        return ref.at[pl.ds(base_w + i * _CH, _CH)]

    idx_cp = [None] * (_NCHUNK + 2)
    g_u = [None] * _NCHUNK
    g_e = [None] * _NCHUNK
    wb_u = [None] * _NCHUNK
    wb_e = [None] * _NCHUNK
    idx_cp[0] = pltpu.async_copy(idx_slice(0), idxv[0], si[0])
    idx_cp[1] = pltpu.async_copy(idx_slice(1), idxv[1], si[1])
    for i in range(_NCHUNK):
        b = i % 2
        idx_cp[i].wait()

        @pl.loop(0, _CH, step=16)
        def _fold(j, b=b):
            v = idxv[b][pl.ds(j, 16)]
            fidx[b][pl.ds(j, 16)] = jnp.where(v < N_USERS, v, v - N_USERS)

        if i + 2 < _NCHUNK:
            idx_cp[i + 2] = pltpu.async_copy(idx_slice(i + 2), idxv[b], si[b])
        if i >= 2:
            wb_u[i - 2].wait()
            wb_e[i - 2].wait()
        g_u[i] = pltpu.async_copy(user_hbm.at[fidx[b]], ru[b], sgu[b])
        g_e[i] = pltpu.async_copy(ent_hbm.at[fidx[b]], re[b], sge[b])
        if i >= 1:
            pb = (i - 1) % 2
            g_u[i - 1].wait()
            g_e[i - 1].wait()
            wb_u[i - 1] = pltpu.async_copy(ru[pb], out_slice(out_u, i - 1),
                                           swu[pb])
            wb_e[i - 1] = pltpu.async_copy(re[pb], out_slice(out_e, i - 1),
                                           swe[pb])
    last = _NCHUNK - 1
    lb = last % 2
    g_u[last].wait()
    g_e[last].wait()
    wb_u[last] = pltpu.async_copy(ru[lb], out_slice(out_u, last), swu[lb])
    wb_e[last] = pltpu.async_copy(re[lb], out_slice(out_e, last), swe[lb])
    wb_u[last - 1].wait()
    wb_e[last - 1].wait()
    wb_u[last].wait()
    wb_e[last].wait()


def _sc_gather(user_embed, entity_embed, idx_all):
    mesh = plsc.VectorSubcoreMesh(core_axis_name="c", subcore_axis_name="s")
    f32 = jnp.float32
    i32 = jnp.int32
    kern = pl.kernel(
        _sc_gather_body,
        mesh=mesh,
        compiler_params=pltpu.CompilerParams(use_tc_tiling_on_sc=False),
        out_type=[jax.ShapeDtypeStruct((_TOT, EMB_DIM), f32),
                  jax.ShapeDtypeStruct((_TOT, EMB_DIM), f32)],
        scratch_types=(
            [pltpu.VMEM((_CH,), i32) for _ in range(4)]
            + [pltpu.VMEM((_CH, EMB_DIM), f32) for _ in range(4)]
            + [pltpu.SemaphoreType.DMA for _ in range(10)]
        ),
    )
    return kern(user_embed, entity_embed, idx_all)


_T = 512
_GRID = B // _T


def _tc_body(hu, he, pu, pe, nu, ne, hidx, pidx, nidx, rel,
             rel_emb, w2, out_ref):
    f32 = jnp.float32
    h = jnp.where(hidx[...] < N_USERS, hu[...], he[...])
    p = jnp.where(pidx[...] < N_USERS, pu[...], pe[...])
    n = jnp.where(nidx[...] < N_USERS, nu[...], ne[...])
    dp = h - p
    dn = h - n
    x = jnp.concatenate([dp, dn], axis=0)
    y = lax.dot_general(x, w2[...], (((1,), (0,)), ((), ())),
                        preferred_element_type=f32,
                        precision=lax.Precision.DEFAULT)
    oh = (rel[...] == lax.broadcasted_iota(jnp.int32, (_T, N_RELATIONS), 1)
          ).astype(f32)
    r_e = lax.dot_general(oh, rel_emb[...], (((1,), (0,)), ((), ())),
                          preferred_element_type=f32,
                          precision=lax.Precision.HIGHEST)
    oh2 = jnp.concatenate([oh, oh], axis=0)
    acc = jnp.zeros((2 * _T, KGE_DIM), f32)
    for r in range(N_RELATIONS):
        acc = acc + y[:, r * KGE_DIM:(r + 1) * KGE_DIM] * oh2[:, r:r + 1]
    sp = acc[:_T] + r_e
    sn = acc[_T:] + r_e
    ps = jnp.sum(sp * sp, axis=1)
    ns = jnp.sum(sn * sn, axis=1)
    d = ps - ns
    part = jnp.sum(jnp.maximum(d, 0.0) + jnp.log1p(jnp.exp(-jnp.abs(d))))

    @pl.when(pl.program_id(0) == 0)
    def _init():
        out_ref[0, 0] = 0.0

    out_ref[0, 0] += part * (1.0 / B)


def _tc_loss(rows_u, rows_e, heads, pos_tails, neg_tails, relations,
             relation_embed, w2, interpret=False):
    f32 = jnp.float32
    row_spec = lambda seg: pl.BlockSpec((_T, EMB_DIM),
                                        lambda i, s=seg: (s * _GRID + i, 0))
    idx_spec = pl.BlockSpec((_T, 1), lambda i: (i, 0))
    full = lambda shape: pl.BlockSpec(shape, lambda i: (0, 0))
    out = pl.pallas_call(
        _tc_body,
        grid=(_GRID,),
        in_specs=[
            row_spec(0), row_spec(0),
            row_spec(1), row_spec(1),
            row_spec(2), row_spec(2),
            idx_spec, idx_spec, idx_spec, idx_spec,
            full((N_RELATIONS, KGE_DIM)),
            full((EMB_DIM, N_RELATIONS * KGE_DIM)),
        ],
        out_specs=pl.BlockSpec(memory_space=pltpu.SMEM),
        out_shape=jax.ShapeDtypeStruct((1, 1), f32),
        interpret=interpret,
    )(rows_u, rows_e, rows_u, rows_e, rows_u, rows_e,
      heads, pos_tails, neg_tails, relations,
      relation_embed, w2)
    return out[0, 0]


def kernel(user_embed, entity_embed, relation_embed, trans_W,
           heads, relations, pos_tails, neg_tails):
    i32 = jnp.int32
    heads = heads.astype(i32)
    pos_tails = pos_tails.astype(i32)
    neg_tails = neg_tails.astype(i32)
    relations = relations.astype(i32)

    idx_all = jnp.concatenate([heads, pos_tails, neg_tails])
    rows_u, rows_e = _sc_gather(user_embed, entity_embed, idx_all)

    w2 = jnp.transpose(trans_W, (1, 0, 2)).reshape(EMB_DIM,
                                                   N_RELATIONS * KGE_DIM)
    return _tc_loss(rows_u, rows_e,
                    heads.reshape(B, 1), pos_tails.reshape(B, 1),
                    neg_tails.reshape(B, 1), relations.reshape(B, 1),
                    relation_embed, w2)
